# baseline (speedup 1.0000x reference)
import functools

import jax
import jax.numpy as jnp
from jax import lax
from jax.experimental import pallas as pl
from jax.experimental.pallas import tpu as pltpu
from jax.experimental.pallas import tpu_sc as plsc

N = 10000
E = 320000
D = 128
H = 128
G = 64
OUT = 24

NP = 10240
NC = 2
NS = 16
NW = NC * NS
CH = 128
IG = 8
RB = 2048
NPW = NP // NW
LFIX = 11264
LC = LFIX // CH


def _make_sc_agg_local():
    mesh = plsc.VectorSubcoreMesh(core_axis_name="c", subcore_axis_name="s")
    scratch = [
        pltpu.VMEM((IG, CH), jnp.int32),
        pltpu.VMEM((IG, CH), jnp.int32),
        pltpu.VMEM((CH, H), jnp.float32),
        pltpu.VMEM((CH, H), jnp.float32),
        pltpu.VMEM_SHARED((NS * NPW, H), jnp.float32),
        pltpu.SemaphoreType.DMA,
        pltpu.SemaphoreType.DMA,
        pltpu.SemaphoreType.DMA,
        pltpu.SemaphoreType.DMA,
    ]

    def body(t_hbm, ls3, ld3, z128, out,
             src_v, dst_v, rows_a, rows_b, acc, gsa, gsb, ssa, ssb):
        c = lax.axis_index("c")
        s = lax.axis_index("s")
        w = s * NC + c
        base = w * NPW
        cbase = s * NPW
        bufs = (rows_a, rows_b)
        gsems = (gsa, gsb)
        ssems = (ssa, ssb)

        pltpu.sync_copy(z128, rows_a)
        pltpu.sync_copy(rows_a, acc.at[pl.ds(cbase, CH)])
        pltpu.sync_copy(rows_a, acc.at[pl.ds(cbase + CH, CH)])
        pltpu.sync_copy(rows_a.at[pl.ds(0, NPW - 2 * CH)],
                        acc.at[pl.ds(cbase + 2 * CH, NPW - 2 * CH)])

        def step(o, carry):
            pltpu.sync_copy(ls3.at[w, pl.ds(o * IG, IG)], src_v)
            pltpu.sync_copy(ld3.at[w, pl.ds(o * IG, IG)], dst_v)
            gd = pltpu.async_copy(t_hbm.at[src_v.at[0]], bufs[0], gsems[0])
            sd = None
            for g in range(IG):
                p, q = g % 2, (g + 1) % 2
                gd.wait()
                if sd is not None:
                    sd.wait()
                if g + 1 < IG:
                    gd = pltpu.async_copy(t_hbm.at[src_v.at[g + 1]],
                                          bufs[q], gsems[q])
                sd = pltpu.async_copy(bufs[p], acc.at[dst_v.at[g]],
                                      ssems[p], add=True)
            sd.wait()
            return carry

        lax.fori_loop(0, LC // IG, step, 0)

        for k, width in ((0, CH), (CH, CH), (2 * CH, NPW - 2 * CH)):
            pltpu.sync_copy(acc.at[pl.ds(cbase + k, width)],
                            rows_a.at[pl.ds(0, width)])
            pltpu.sync_copy(rows_a.at[pl.ds(0, width)],
                            out.at[pl.ds(base + k, width)])

    return pl.kernel(body,
                     out_type=jax.ShapeDtypeStruct((NP, H), jnp.float32),
                     mesh=mesh, scratch_types=scratch)


_sc_agg_local = _make_sc_agg_local()


def _row_spec(width):
    return pl.BlockSpec((RB, width), lambda i: (i, 0))


def _full_spec(shape):
    return pl.BlockSpec(shape, lambda i: (0, 0))


def _pad_mask(pid):
    rows = lax.broadcasted_iota(jnp.int32, (RB, H), 0) + pid * RB
    return (rows < N).astype(jnp.float32)


def _tc_first(x, wn):
    def body(x_ref, w_ref, t_ref):
        t_ref[...] = jnp.dot(x_ref[...], w_ref[...],
                             preferred_element_type=jnp.float32)

    return pl.pallas_call(
        body,
        grid=(NP // RB,),
        in_specs=[_row_spec(D), _full_spec((D, H))],
        out_specs=_row_spec(H),
        out_shape=jax.ShapeDtypeStruct((NP, H), jnp.float32),
    )(x, wn)


def _tc_update_first(agg, cnt, h, wr, b, wn_next):
    def body(a_ref, c_ref, h_ref, wr_ref, b_ref, wn_ref,
             h_out, t_out, inv_out):
        hn = jnp.maximum(
            a_ref[...] + jnp.dot(h_ref[...], wr_ref[...],
                                 preferred_element_type=jnp.float32)
            + b_ref[...],
            0.0)
        h_out[...] = hn
        t_out[...] = jnp.dot(hn, wn_ref[...],
                             preferred_element_type=jnp.float32) \
            * _pad_mask(pl.program_id(0))
        inv_out[...] = 1.0 / jnp.maximum(c_ref[:, 0:1], 1.0)

    return pl.pallas_call(
        body,
        grid=(NP // RB,),
        in_specs=[_row_spec(H), _row_spec(H),
                  _row_spec(D), _full_spec((D, H)), _full_spec((1, H)),
                  _full_spec((H, H))],
        out_specs=(_row_spec(H), _row_spec(H), _row_spec(1)),
        out_shape=(jax.ShapeDtypeStruct((NP, H), jnp.float32),
                   jax.ShapeDtypeStruct((NP, H), jnp.float32),
                   jax.ShapeDtypeStruct((NP, 1), jnp.float32)),
    )(agg, cnt, h, wr, b, wn_next)


def _tc_update_mean(agg, inv, h, wr, b, wn_next):
    def body(a_ref, inv_ref, h_ref, wr_ref, b_ref, wn_ref, h_out, t_out):
        hn = jnp.maximum(
            a_ref[...] * inv_ref[...]
            + jnp.dot(h_ref[...], wr_ref[...],
                      preferred_element_type=jnp.float32) + b_ref[...],
            0.0)
        h_out[...] = hn
        t_out[...] = jnp.dot(hn, wn_ref[...],
                             preferred_element_type=jnp.float32) \
            * _pad_mask(pl.program_id(0))

    return pl.pallas_call(
        body,
        grid=(NP // RB,),
        in_specs=[_row_spec(H), _row_spec(1), _row_spec(H),
                  _full_spec((H, H)), _full_spec((1, H)), _full_spec((H, H))],
        out_specs=(_row_spec(H), _row_spec(H)),
        out_shape=(jax.ShapeDtypeStruct((NP, H), jnp.float32),
                   jax.ShapeDtypeStruct((NP, H), jnp.float32)),
    )(agg, inv, h, wr, b, wn_next)


def _tc_update_last(agg, inv, h, wr, b):
    def body(a_ref, inv_ref, h_ref, wr_ref, b_ref, h_out):
        h_out[...] = jnp.maximum(
            a_ref[...] * inv_ref[...]
            + jnp.dot(h_ref[...], wr_ref[...],
                      preferred_element_type=jnp.float32) + b_ref[...],
            0.0)

    return pl.pallas_call(
        body,
        grid=(NP // RB,),
        in_specs=[_row_spec(H), _row_spec(1), _row_spec(H),
                  _full_spec((H, H)), _full_spec((1, H))],
        out_specs=_row_spec(H),
        out_shape=jax.ShapeDtypeStruct((NP, H), jnp.float32),
    )(agg, inv, h, wr, b)


def _tc_pool(h, batch2, lin_w, lin_b):
    def body(h_ref, b_ref, w_ref, lb_ref, o_ref):
        gids = lax.broadcasted_iota(jnp.int32, (G, NP), 0)
        onehot = (gids == b_ref[...]).astype(jnp.float32)
        pooled = jnp.dot(onehot, h_ref[...], preferred_element_type=jnp.float32)
        cnt = jnp.sum(onehot, axis=1, keepdims=True)
        o_ref[...] = (jnp.dot(pooled / jnp.maximum(cnt, 1.0), w_ref[...],
                              preferred_element_type=jnp.float32)
                      + lb_ref[...])

    return pl.pallas_call(
        body,
        in_specs=[pl.BlockSpec((NP, H), lambda: (0, 0)),
                  pl.BlockSpec((1, NP), lambda: (0, 0)),
                  pl.BlockSpec((H, OUT), lambda: (0, 0)),
                  pl.BlockSpec((1, OUT), lambda: (0, 0))],
        out_specs=pl.BlockSpec((G, OUT), lambda: (0, 0)),
        out_shape=jax.ShapeDtypeStruct((G, OUT), jnp.float32),
    )(h, batch2, lin_w, lin_b)


def kernel(x, edge_index, batch,
           conv1_Wn, conv1_Wr, conv1_b,
           conv2_Wn, conv2_Wr, conv2_b,
           conv3_Wn, conv3_Wr, conv3_b,
           conv4_Wn, conv4_Wr, conv4_b,
           conv5_Wn, conv5_Wr, conv5_b,
           conv6_Wn, conv6_Wr, conv6_b,
           conv7_Wn, conv7_Wr, conv7_b,
           lin_W, lin_b):
    wns = [conv1_Wn, conv2_Wn, conv3_Wn, conv4_Wn, conv5_Wn, conv6_Wn, conv7_Wn]
    wrs = [conv1_Wr, conv2_Wr, conv3_Wr, conv4_Wr, conv5_Wr, conv6_Wr, conv7_Wr]
    bs = [b.reshape(1, H) for b in
          [conv1_b, conv2_b, conv3_b, conv4_b, conv5_b, conv6_b, conv7_b]]

    xp = jnp.pad(x, ((0, NP - N), (0, 0)))
    batch2 = jnp.pad(batch, (0, NP - N), constant_values=127).reshape(1, NP)

    z128 = jnp.zeros((CH, H), jnp.float32)
    ones_t = jnp.pad(jnp.ones((N, H), jnp.float32), ((0, NP - N), (0, 0)))

    src, dst = edge_index[0], edge_index[1]
    order = jnp.argsort(dst)
    src_s = src[order]
    dst_s = dst[order]
    w_s = dst_s // NPW
    starts = jnp.searchsorted(dst_s, jnp.arange(NW, dtype=jnp.int32) * NPW)
    pos = jnp.arange(E, dtype=jnp.int32) - starts[w_s]
    flat = jnp.where(pos < LFIX, w_s * LFIX + pos, NW * LFIX)
    dl_s = dst_s - w_s * NPW + (w_s // NC) * NPW
    ls = jnp.full((NW * LFIX + 1,), N, jnp.int32).at[flat].set(src_s)
    ld_fill = jnp.repeat((jnp.arange(NW, dtype=jnp.int32) // NC) * NPW, LFIX)
    ld = jnp.concatenate([ld_fill, jnp.zeros((1,), jnp.int32)]).at[flat].set(dl_s)
    ls3 = ls[:NW * LFIX].reshape(NW, LC, CH)
    ld3 = ld[:NW * LFIX].reshape(NW, LC, CH)

    t = _tc_first(xp, wns[0])
    cnt = _sc_agg_local(ones_t, ls3, ld3, z128)
    agg = _sc_agg_local(t, ls3, ld3, z128)
    h, t, inv = _tc_update_first(agg, cnt, xp, wrs[0], bs[0], wns[1])
    for i in range(1, 6):
        agg = _sc_agg_local(t, ls3, ld3, z128)
        h, t = _tc_update_mean(agg, inv, h, wrs[i], bs[i], wns[i + 1])
    agg = _sc_agg_local(t, ls3, ld3, z128)
    h = _tc_update_last(agg, inv, h, wrs[6], bs[6])

    return _tc_pool(h, batch2, lin_W, lin_b.reshape(1, OUT))


# speedup vs baseline: 5.1121x; 5.1121x over previous
import jax
import jax.numpy as jnp
from jax import lax
from jax.experimental import pallas as pl
from jax.experimental.pallas import tpu as pltpu
from jax.experimental.pallas import tpu_sc as plsc

N = 10000
E = 320000
D = 128
H = 128
G = 64
OUT = 24

NP = 10240
NC = 2
NS = 16
NW = NC * NS
CH = 128
EW = 10240
CG = EW // CH
IG = 8
OG = CG // IG
RPT = NP // NS
RB = 2048


def _make_sc_agg():
    mesh = plsc.VectorSubcoreMesh(core_axis_name="c", subcore_axis_name="s")
    scratch = [
        pltpu.VMEM((IG, CH), jnp.int32),
        pltpu.VMEM((IG, CH), jnp.int32),
        pltpu.VMEM((CH, H), jnp.float32),
        pltpu.VMEM((CH, H), jnp.float32),
        pltpu.VMEM_SHARED((NP, H), jnp.float32),
        pltpu.SemaphoreType.DMA,
        pltpu.SemaphoreType.DMA,
        pltpu.SemaphoreType.DMA,
        pltpu.SemaphoreType.DMA,
    ]

    def body(t_hbm, srcb, dstb, z128, part_out,
             src_v, dst_v, rows_a, rows_b, acc_sh, gsa, gsb, ssa, ssb):
        c = lax.axis_index("c")
        s = lax.axis_index("s")
        w = s * NC + c
        base = s * RPT
        bufs = (rows_a, rows_b)
        gsems = (gsa, gsb)
        ssems = (ssa, ssb)

        pltpu.sync_copy(z128, rows_a)
        for k in range(RPT // CH):
            pltpu.sync_copy(rows_a, acc_sh.at[pl.ds(base + k * CH, CH)])
        plsc.subcore_barrier()

        def step(o, carry):
            pltpu.sync_copy(srcb.at[w, pl.ds(o * IG, IG)], src_v)
            pltpu.sync_copy(dstb.at[w, pl.ds(o * IG, IG)], dst_v)
            gd = pltpu.async_copy(t_hbm.at[src_v.at[0]], bufs[0], gsems[0])
            sd = None
            for g in range(IG):
                p, q = g % 2, (g + 1) % 2
                gd.wait()
                if sd is not None:
                    sd.wait()
                if g + 1 < IG:
                    gd = pltpu.async_copy(t_hbm.at[src_v.at[g + 1]],
                                          bufs[q], gsems[q])
                sd = pltpu.async_copy(bufs[p], acc_sh.at[dst_v.at[g]],
                                      ssems[p], add=True)
            sd.wait()
            return carry

        lax.fori_loop(0, OG, step, 0)
        plsc.subcore_barrier()

        for k in range(RPT // CH):
            pltpu.sync_copy(acc_sh.at[pl.ds(base + k * CH, CH)], rows_a)
            pltpu.sync_copy(rows_a, part_out.at[c, pl.ds(base + k * CH, CH)])

    return pl.kernel(body,
                     out_type=jax.ShapeDtypeStruct((NC, NP, H), jnp.float32),
                     mesh=mesh, scratch_types=scratch)


def _make_sc_cnt():
    mesh = plsc.VectorSubcoreMesh(core_axis_name="c", subcore_axis_name="s")
    scratch = [
        pltpu.VMEM((IG, CH), jnp.int32),
        pltpu.VMEM((CH, H), jnp.float32),
        pltpu.VMEM_SHARED((NP, H), jnp.float32),
        pltpu.SemaphoreType.DMA,
    ]

    def body(dstb, z128, o128, cnt_out, dst_v, c_v, cnt_sh, sem):
        c = lax.axis_index("c")
        s = lax.axis_index("s")
        w = s * NC + c
        base = s * RPT

        pltpu.sync_copy(z128, c_v)
        for k in range(RPT // CH):
            pltpu.sync_copy(c_v, cnt_sh.at[pl.ds(base + k * CH, CH)])
        pltpu.sync_copy(o128, c_v)
        plsc.subcore_barrier()

        def step(o, carry):
            pltpu.sync_copy(dstb.at[w, pl.ds(o * IG, IG)], dst_v)
            descs = [pltpu.async_copy(c_v, cnt_sh.at[dst_v.at[g]], sem,
                                      add=True)
                     for g in range(IG)]
            for d in descs:
                d.wait()
            return carry

        lax.fori_loop(0, OG, step, 0)
        plsc.subcore_barrier()

        for k in range(RPT // CH):
            pltpu.sync_copy(cnt_sh.at[pl.ds(base + k * CH, CH)], c_v)
            pltpu.sync_copy(c_v, cnt_out.at[c, pl.ds(base + k * CH, CH)])

    return pl.kernel(body,
                     out_type=jax.ShapeDtypeStruct((NC, NP, H), jnp.float32),
                     mesh=mesh, scratch_types=scratch)


_sc_agg = _make_sc_agg()
_sc_cnt = _make_sc_cnt()


def _row_spec(width):
    return pl.BlockSpec((RB, width), lambda i: (i, 0))


def _full_spec(shape):
    return pl.BlockSpec(shape, lambda i: (0, 0))


def _tc_first(x, wn):
    def body(x_ref, w_ref, t_ref):
        t_ref[...] = jnp.dot(x_ref[...], w_ref[...],
                             preferred_element_type=jnp.float32)

    return pl.pallas_call(
        body,
        grid=(NP // RB,),
        in_specs=[_row_spec(D), _full_spec((D, H))],
        out_specs=_row_spec(H),
        out_shape=jax.ShapeDtypeStruct((NP, H), jnp.float32),
    )(x, wn)


def _tc_update_first(p0, p1, c0, c1, h, wr, b, wn_next):
    def body(p0_ref, p1_ref, c0_ref, c1_ref, h_ref, wr_ref, b_ref, wn_ref,
             h_out, t_out, inv_out):
        agg = p0_ref[...] + p1_ref[...]
        hn = jnp.maximum(
            agg + jnp.dot(h_ref[...], wr_ref[...],
                          preferred_element_type=jnp.float32) + b_ref[...],
            0.0)
        h_out[...] = hn
        t_out[...] = jnp.dot(hn, wn_ref[...], preferred_element_type=jnp.float32)
        cnt = c0_ref[:, 0:1] + c1_ref[:, 0:1]
        inv_out[...] = 1.0 / jnp.maximum(cnt, 1.0)

    return pl.pallas_call(
        body,
        grid=(NP // RB,),
        in_specs=[_row_spec(H), _row_spec(H), _row_spec(H), _row_spec(H),
                  _row_spec(D), _full_spec((D, H)), _full_spec((1, H)),
                  _full_spec((H, H))],
        out_specs=(_row_spec(H), _row_spec(H), _row_spec(1)),
        out_shape=(jax.ShapeDtypeStruct((NP, H), jnp.float32),
                   jax.ShapeDtypeStruct((NP, H), jnp.float32),
                   jax.ShapeDtypeStruct((NP, 1), jnp.float32)),
    )(p0, p1, c0, c1, h, wr, b, wn_next)


def _tc_update_mean(p0, p1, inv, h, wr, b, wn_next):
    def body(p0_ref, p1_ref, inv_ref, h_ref, wr_ref, b_ref, wn_ref,
             h_out, t_out):
        agg = (p0_ref[...] + p1_ref[...]) * inv_ref[...]
        hn = jnp.maximum(
            agg + jnp.dot(h_ref[...], wr_ref[...],
                          preferred_element_type=jnp.float32) + b_ref[...],
            0.0)
        h_out[...] = hn
        t_out[...] = jnp.dot(hn, wn_ref[...], preferred_element_type=jnp.float32)

    return pl.pallas_call(
        body,
        grid=(NP // RB,),
        in_specs=[_row_spec(H), _row_spec(H), _row_spec(1), _row_spec(H),
                  _full_spec((H, H)), _full_spec((1, H)), _full_spec((H, H))],
        out_specs=(_row_spec(H), _row_spec(H)),
        out_shape=(jax.ShapeDtypeStruct((NP, H), jnp.float32),
                   jax.ShapeDtypeStruct((NP, H), jnp.float32)),
    )(p0, p1, inv, h, wr, b, wn_next)


def _tc_update_last(p0, p1, inv, h, wr, b):
    def body(p0_ref, p1_ref, inv_ref, h_ref, wr_ref, b_ref, h_out):
        agg = (p0_ref[...] + p1_ref[...]) * inv_ref[...]
        h_out[...] = jnp.maximum(
            agg + jnp.dot(h_ref[...], wr_ref[...],
                          preferred_element_type=jnp.float32) + b_ref[...],
            0.0)

    return pl.pallas_call(
        body,
        grid=(NP // RB,),
        in_specs=[_row_spec(H), _row_spec(H), _row_spec(1), _row_spec(H),
                  _full_spec((H, H)), _full_spec((1, H))],
        out_specs=_row_spec(H),
        out_shape=jax.ShapeDtypeStruct((NP, H), jnp.float32),
    )(p0, p1, inv, h, wr, b)


def _tc_pool(h, batch2, lin_w, lin_b):
    def body(h_ref, b_ref, w_ref, lb_ref, o_ref):
        gids = lax.broadcasted_iota(jnp.int32, (G, NP), 0)
        onehot = (gids == b_ref[...]).astype(jnp.float32)
        pooled = jnp.dot(onehot, h_ref[...], preferred_element_type=jnp.float32)
        cnt = jnp.sum(onehot, axis=1, keepdims=True)
        o_ref[...] = (jnp.dot(pooled / jnp.maximum(cnt, 1.0), w_ref[...],
                              preferred_element_type=jnp.float32)
                      + lb_ref[...])

    return pl.pallas_call(
        body,
        in_specs=[pl.BlockSpec((NP, H), lambda: (0, 0)),
                  pl.BlockSpec((1, NP), lambda: (0, 0)),
                  pl.BlockSpec((H, OUT), lambda: (0, 0)),
                  pl.BlockSpec((1, OUT), lambda: (0, 0))],
        out_specs=pl.BlockSpec((G, OUT), lambda: (0, 0)),
        out_shape=jax.ShapeDtypeStruct((G, OUT), jnp.float32),
    )(h, batch2, lin_w, lin_b)


def kernel(x, edge_index, batch,
           conv1_Wn, conv1_Wr, conv1_b,
           conv2_Wn, conv2_Wr, conv2_b,
           conv3_Wn, conv3_Wr, conv3_b,
           conv4_Wn, conv4_Wr, conv4_b,
           conv5_Wn, conv5_Wr, conv5_b,
           conv6_Wn, conv6_Wr, conv6_b,
           conv7_Wn, conv7_Wr, conv7_b,
           lin_W, lin_b):
    wns = [conv1_Wn, conv2_Wn, conv3_Wn, conv4_Wn, conv5_Wn, conv6_Wn, conv7_Wn]
    wrs = [conv1_Wr, conv2_Wr, conv3_Wr, conv4_Wr, conv5_Wr, conv6_Wr, conv7_Wr]
    bs = [b.reshape(1, H) for b in
          [conv1_b, conv2_b, conv3_b, conv4_b, conv5_b, conv6_b, conv7_b]]

    xp = jnp.pad(x, ((0, NP - N), (0, 0)))
    batch2 = jnp.pad(batch, (0, NP - N), constant_values=127).reshape(1, NP)

    epw = E // NW
    src = edge_index[0].reshape(NW, epw)
    dst = edge_index[1].reshape(NW, epw)
    pad = jnp.full((NW, EW - epw), N, jnp.int32)
    srcb = jnp.concatenate([src, pad], axis=1).reshape(NW, CG, CH)
    dstb = jnp.concatenate([dst, pad], axis=1).reshape(NW, CG, CH)

    z128 = jnp.zeros((CH, H), jnp.float32)
    o128 = jnp.ones((CH, H), jnp.float32)

    t = _tc_first(xp, wns[0])
    cnts = _sc_cnt(dstb, z128, o128)
    parts = _sc_agg(t, srcb, dstb, z128)
    h, t, inv = _tc_update_first(parts[0], parts[1], cnts[0], cnts[1],
                                 xp, wrs[0], bs[0], wns[1])
    for i in range(1, 6):
        parts = _sc_agg(t, srcb, dstb, z128)
        h, t = _tc_update_mean(parts[0], parts[1], inv, h, wrs[i], bs[i],
                               wns[i + 1])
    parts = _sc_agg(t, srcb, dstb, z128)
    h = _tc_update_last(parts[0], parts[1], inv, h, wrs[6], bs[6])

    return _tc_pool(h, batch2, lin_W, lin_b.reshape(1, OUT))
